# initial kernel scaffold (unmeasured)
import jax
import jax.numpy as jnp
from jax import lax
from jax.experimental import pallas as pl
from jax.experimental.pallas import tpu as pltpu

N_DEV = 16
B, Sq, D, Hq, Dh = 4, 256, 1024, 8, 128
SCALE = 0.08838834764831843


def kernel(x, Wq, Wo, K_ext, V_ext):
    def body(x_ref, wq_ref, wo_ref, k_ref, v_ref, out_ref,
             acc_ref, comm_ref, send_sems, recv_sems):
        my = lax.axis_index("i")
        left = (my - 1) % N_DEV
        right = (my + 1) % N_DEV

        barrier = pltpu.get_barrier_semaphore()
        for nbr in (left, right):
            pl.semaphore_signal(barrier, inc=1, device_id=(nbr,),
                                device_id_type=pl.DeviceIdType.MESH)
        pl.semaphore_wait(barrier, 2)

        for b in range(B):
            for h in range(Hq):
                q = jnp.dot(x_ref[b], wq_ref[:, h * Dh:(h + 1) * Dh],
                            preferred_element_type=jnp.float32)
                s = lax.dot_general(q, k_ref[b, :, h, :],
                                    (((1,), (1,)), ((), ())),
                                    preferred_element_type=jnp.float32) * SCALE
                p = jnp.exp(s)
                o = jnp.dot(p, v_ref[b, :, h, :],
                            preferred_element_type=jnp.float32)
                comm_ref[0, b, h, :, 0:Dh] = o
                comm_ref[0, b, h, :, Dh:Dh + 1] = jnp.sum(
                    p, axis=1, keepdims=True)

        acc_ref[...] = comm_ref[0]

        for hop in range(N_DEV - 1):
            send_slot = hop % 2
            recv_slot = (hop + 1) % 2
            rdma = pltpu.make_async_remote_copy(
                src_ref=comm_ref.at[send_slot],
                dst_ref=comm_ref.at[recv_slot],
                send_sem=send_sems.at[send_slot],
                recv_sem=recv_sems.at[recv_slot],
                device_id=(right,),
                device_id_type=pl.DeviceIdType.MESH,
            )
            rdma.start()
            rdma.wait()
            acc_ref[...] = acc_ref[...] + comm_ref[recv_slot]

        for b in range(B):
            acc_out = jnp.zeros((Sq, D), jnp.float32)
            for h in range(Hq):
                o = acc_ref[b, h, :, 0:Dh]
                l = acc_ref[b, h, :, Dh:Dh + 1]
                acc_out = acc_out + jnp.dot(
                    o / l, wo_ref[h * Dh:(h + 1) * Dh, :],
                    preferred_element_type=jnp.float32)
            out_ref[b] = acc_out

    return pl.pallas_call(
        body,
        out_shape=jax.ShapeDtypeStruct((B, Sq, D), jnp.float32),
        in_specs=[pl.BlockSpec(memory_space=pltpu.VMEM)] * 5,
        out_specs=pl.BlockSpec(memory_space=pltpu.VMEM),
        scratch_shapes=[
            pltpu.VMEM((B, Hq, Sq, Dh + 1), jnp.float32),
            pltpu.VMEM((2, B, Hq, Sq, Dh + 1), jnp.float32),
            pltpu.SemaphoreType.DMA((2,)),
            pltpu.SemaphoreType.DMA((2,)),
        ],
        compiler_params=pltpu.CompilerParams(
            collective_id=0,
            vmem_limit_bytes=128 * 1024 * 1024,
        ),
    )(x, Wq, Wo, K_ext, V_ext)


# baseline (device time: 852775 ns/iter reference)
import jax
import jax.numpy as jnp
from jax import lax
from jax.experimental import pallas as pl
from jax.experimental.pallas import tpu as pltpu

N_DEV = 16
B, Sq, D, Hq, Dh = 4, 256, 1024, 8, 128
SKV = 1024
SCALE = 0.08838834764831843
HL = Hq + 1


def kernel(x, Wq, Wo, K_ext, V_ext):
    def body(x_ref, wq_ref, wo_ref, k_ref, v_ref, out_ref,
             qs_ref, n_ref, acc_ref, comm_ref, kbuf, vbuf,
             send_sems, recv_sems, local_sems):
        my = lax.axis_index("i")
        left = (my - 1) % N_DEV
        right = (my + 1) % N_DEV

        barrier = pltpu.get_barrier_semaphore()
        for nbr in (left, right):
            pl.semaphore_signal(barrier, inc=1, device_id=(nbr,),
                                device_id_type=pl.DeviceIdType.MESH)
        pl.semaphore_wait(barrier, 2)

        qs_ref[...] = jnp.dot(
            x_ref[...].reshape(B * Sq, D), wq_ref[...],
            preferred_element_type=jnp.float32) * SCALE

        def attn_b(b, carry):
            cp_k = pltpu.make_async_copy(k_ref.at[b], kbuf, local_sems.at[0])
            cp_v = pltpu.make_async_copy(v_ref.at[b], vbuf, local_sems.at[1])
            cp_k.start()
            cp_v.start()
            cp_k.wait()
            cp_v.wait()
            for h in range(Hq):
                q = qs_ref[pl.ds(b * Sq, Sq), h * Dh:(h + 1) * Dh]
                s = lax.dot_general(q, kbuf[:, h, :],
                                    (((1,), (1,)), ((), ())),
                                    preferred_element_type=jnp.float32)
                p = jnp.exp(s)
                comm_ref[0, b, h] = jnp.dot(
                    p, vbuf[:, h, :], preferred_element_type=jnp.float32)
                comm_ref[0, b, Hq, :, h:h + 1] = jnp.sum(
                    p, axis=1, keepdims=True)
            return carry

        lax.fori_loop(0, B, attn_b, 0)

        acc_ref[...] = comm_ref[0]

        for hop in range(N_DEV - 1):
            send_slot = hop % 2
            recv_slot = (hop + 1) % 2
            rdma = pltpu.make_async_remote_copy(
                src_ref=comm_ref.at[send_slot],
                dst_ref=comm_ref.at[recv_slot],
                send_sem=send_sems.at[send_slot],
                recv_sem=recv_sems.at[recv_slot],
                device_id=(right,),
                device_id_type=pl.DeviceIdType.MESH,
            )
            rdma.start()
            rdma.wait()
            acc_ref[...] = acc_ref[...] + comm_ref[recv_slot]

        def norm_b(b, carry):
            for h in range(Hq):
                n_ref[pl.ds(b * Sq, Sq), h * Dh:(h + 1) * Dh] = (
                    acc_ref[b, h] / acc_ref[b, Hq, :, h:h + 1])
            return carry

        lax.fori_loop(0, B, norm_b, 0)

        res = jnp.dot(n_ref[...], wo_ref[...],
                      preferred_element_type=jnp.float32)
        for b in range(B):
            out_ref[b] = res[b * Sq:(b + 1) * Sq, :]

    return pl.pallas_call(
        body,
        out_shape=jax.ShapeDtypeStruct((B, Sq, D), jnp.float32),
        in_specs=[
            pl.BlockSpec(memory_space=pltpu.MemorySpace.VMEM),
            pl.BlockSpec(memory_space=pltpu.MemorySpace.VMEM),
            pl.BlockSpec(memory_space=pltpu.MemorySpace.VMEM),
            pl.BlockSpec(memory_space=pltpu.MemorySpace.HBM),
            pl.BlockSpec(memory_space=pltpu.MemorySpace.HBM),
        ],
        out_specs=pl.BlockSpec(memory_space=pltpu.MemorySpace.VMEM),
        scratch_shapes=[
            pltpu.VMEM((B * Sq, D), jnp.float32),
            pltpu.VMEM((B * Sq, D), jnp.float32),
            pltpu.VMEM((B, HL, Sq, Dh), jnp.float32),
            pltpu.VMEM((2, B, HL, Sq, Dh), jnp.float32),
            pltpu.VMEM((SKV, Hq, Dh), jnp.float32),
            pltpu.VMEM((SKV, Hq, Dh), jnp.float32),
            pltpu.SemaphoreType.DMA((2,)),
            pltpu.SemaphoreType.DMA((2,)),
            pltpu.SemaphoreType.DMA((2,)),
        ],
        compiler_params=pltpu.CompilerParams(
            collective_id=0,
            vmem_limit_bytes=128 * 1024 * 1024,
        ),
    )(x, Wq, Wo, K_ext, V_ext)


# device time: 200328 ns/iter; 4.2569x vs baseline; 4.2569x over previous
import jax
import jax.numpy as jnp
from jax import lax
from jax.experimental import pallas as pl
from jax.experimental.pallas import tpu as pltpu

N_DEV = 16
B, Sq, D, Hq, Dh = 4, 256, 1024, 8, 128
SKV = 1024
SCALE = 0.08838834764831843
HL = Hq + 1
CH = Sq // N_DEV


def kernel(x, Wq, Wo, K_ext, V_ext):
    def body(x_ref, wq_ref, wo_ref, k_ref, v_ref, out_ref,
             qs_ref, part_ref, nck_ref, comm_ref, comm2_ref, kbuf, vbuf,
             send_sems, recv_sems, send_sems2, recv_sems2, local_sems):
        my = lax.axis_index("i")
        left = (my - 1) % N_DEV
        right = (my + 1) % N_DEV

        barrier = pltpu.get_barrier_semaphore()
        for nbr in (left, right):
            pl.semaphore_signal(barrier, inc=1, device_id=(nbr,),
                                device_id_type=pl.DeviceIdType.MESH)
        pl.semaphore_wait(barrier, 2)

        qs_ref[...] = jnp.dot(
            x_ref[...].reshape(B * Sq, D), wq_ref[...],
            preferred_element_type=jnp.float32) * SCALE

        def attn_b(b, carry):
            cp_k = pltpu.make_async_copy(k_ref.at[b], kbuf, local_sems.at[0])
            cp_v = pltpu.make_async_copy(v_ref.at[b], vbuf, local_sems.at[1])
            cp_k.start()
            cp_v.start()
            cp_k.wait()
            cp_v.wait()
            for h in range(Hq):
                q = qs_ref[pl.ds(b * Sq, Sq), h * Dh:(h + 1) * Dh]
                s = lax.dot_general(q, kbuf[:, h, :],
                                    (((1,), (1,)), ((), ())),
                                    preferred_element_type=jnp.float32)
                p = jnp.exp(s)
                part_ref[b, h] = jnp.dot(
                    p, vbuf[:, h, :], preferred_element_type=jnp.float32)
                part_ref[b, Hq, :, h:h + 1] = jnp.sum(
                    p, axis=1, keepdims=True)
            return carry

        lax.fori_loop(0, B, attn_b, 0)

        def chunk(c):
            return part_ref[:, :, pl.ds(c * CH, CH), :]

        comm_ref[0] = chunk(my % N_DEV)
        for t in range(N_DEV - 1):
            send_slot = t % 2
            recv_slot = (t + 1) % 2
            rdma = pltpu.make_async_remote_copy(
                src_ref=comm_ref.at[send_slot],
                dst_ref=comm_ref.at[recv_slot],
                send_sem=send_sems.at[send_slot],
                recv_sem=recv_sems.at[recv_slot],
                device_id=(right,),
                device_id_type=pl.DeviceIdType.MESH,
            )
            rdma.start()
            rdma.wait()
            c_in = (my - 1 - t) % N_DEV
            comm_ref[recv_slot] = comm_ref[recv_slot] + chunk(c_in)

        fin = (N_DEV - 1) % 2
        for b in range(B):
            for h in range(Hq):
                nck_ref[b * CH:(b + 1) * CH, h * Dh:(h + 1) * Dh] = (
                    comm_ref[fin, b, h] / comm_ref[fin, b, Hq, :, h:h + 1])
        comm2_ref[0] = jnp.dot(nck_ref[...], wo_ref[...],
                               preferred_element_type=jnp.float32)
        c_my = (my + 1) % N_DEV
        for b in range(B):
            out_ref[b, pl.ds(c_my * CH, CH), :] = comm2_ref[
                0, b * CH:(b + 1) * CH, :]

        for g in range(N_DEV - 1):
            send_slot = g % 2
            recv_slot = (g + 1) % 2
            rdma = pltpu.make_async_remote_copy(
                src_ref=comm2_ref.at[send_slot],
                dst_ref=comm2_ref.at[recv_slot],
                send_sem=send_sems2.at[send_slot],
                recv_sem=recv_sems2.at[recv_slot],
                device_id=(right,),
                device_id_type=pl.DeviceIdType.MESH,
            )
            rdma.start()
            rdma.wait()
            c_o = (my - g) % N_DEV
            for b in range(B):
                out_ref[b, pl.ds(c_o * CH, CH), :] = comm2_ref[
                    recv_slot, b * CH:(b + 1) * CH, :]

    return pl.pallas_call(
        body,
        out_shape=jax.ShapeDtypeStruct((B, Sq, D), jnp.float32),
        in_specs=[
            pl.BlockSpec(memory_space=pltpu.MemorySpace.VMEM),
            pl.BlockSpec(memory_space=pltpu.MemorySpace.VMEM),
            pl.BlockSpec(memory_space=pltpu.MemorySpace.VMEM),
            pl.BlockSpec(memory_space=pltpu.MemorySpace.HBM),
            pl.BlockSpec(memory_space=pltpu.MemorySpace.HBM),
        ],
        out_specs=pl.BlockSpec(memory_space=pltpu.MemorySpace.VMEM),
        scratch_shapes=[
            pltpu.VMEM((B * Sq, D), jnp.float32),
            pltpu.VMEM((B, HL, Sq, Dh), jnp.float32),
            pltpu.VMEM((B * CH, D), jnp.float32),
            pltpu.VMEM((2, B, HL, CH, Dh), jnp.float32),
            pltpu.VMEM((2, B * CH, D), jnp.float32),
            pltpu.VMEM((SKV, Hq, Dh), jnp.float32),
            pltpu.VMEM((SKV, Hq, Dh), jnp.float32),
            pltpu.SemaphoreType.DMA((2,)),
            pltpu.SemaphoreType.DMA((2,)),
            pltpu.SemaphoreType.DMA((2,)),
            pltpu.SemaphoreType.DMA((2,)),
            pltpu.SemaphoreType.DMA((2,)),
        ],
        compiler_params=pltpu.CompilerParams(
            collective_id=0,
            vmem_limit_bytes=128 * 1024 * 1024,
        ),
    )(x, Wq, Wo, K_ext, V_ext)


# device time: 190154 ns/iter; 4.4847x vs baseline; 1.0535x over previous
import jax
import jax.numpy as jnp
from jax import lax
from jax.experimental import pallas as pl
from jax.experimental.pallas import tpu as pltpu

N_DEV = 16
B, Sq, D, Hq, Dh = 4, 256, 1024, 8, 128
SKV = 1024
SCALE = 0.08838834764831843
HL = Hq + 1
CH = Sq // N_DEV


def kernel(x, Wq, Wo, K_ext, V_ext):
    def body(x_ref, wq_ref, wo_ref, k_ref, v_ref, out_ref,
             qs_ref, part_ref, nck_ref, comm_ref, comm2_ref, kbuf, vbuf,
             rs_send, rs_recv, ag_send, ag_recv, local_sems):
        my = lax.axis_index("i")
        left = (my - 1) % N_DEV
        right = (my + 1) % N_DEV

        barrier = pltpu.get_barrier_semaphore()
        for nbr in (left, right):
            pl.semaphore_signal(barrier, inc=1, device_id=(nbr,),
                                device_id_type=pl.DeviceIdType.MESH)
        pl.semaphore_wait(barrier, 2)

        def kv_copies(b, slot):
            cp_k = pltpu.make_async_copy(
                k_ref.at[b], kbuf.at[slot], local_sems.at[slot, 0])
            cp_v = pltpu.make_async_copy(
                v_ref.at[b], vbuf.at[slot], local_sems.at[slot, 1])
            return cp_k, cp_v

        def start_kv(b, slot):
            cp_k, cp_v = kv_copies(b, slot)
            cp_k.start()
            cp_v.start()

        start_kv(0, 0)

        qs_ref[...] = jnp.dot(
            x_ref[...].reshape(B * Sq, D), wq_ref[...],
            preferred_element_type=jnp.float32) * SCALE

        def attn_b(b, carry):
            slot = b % 2

            @pl.when(b + 1 < B)
            def _():
                start_kv(b + 1, (b + 1) % 2)

            cp_k, cp_v = kv_copies(0, slot)
            cp_k.wait()
            cp_v.wait()
            for h in range(Hq):
                q = qs_ref[pl.ds(b * Sq, Sq), h * Dh:(h + 1) * Dh]
                s = lax.dot_general(q, kbuf[slot, :, h, :],
                                    (((1,), (1,)), ((), ())),
                                    preferred_element_type=jnp.float32)
                p = jnp.exp(s)
                part_ref[b, h] = jnp.dot(
                    p, vbuf[slot, :, h, :],
                    preferred_element_type=jnp.float32)
                part_ref[b, Hq, :, h:h + 1] = jnp.sum(
                    p, axis=1, keepdims=True)
            return carry

        lax.fori_loop(0, B, attn_b, 0)

        def chunk(c):
            return part_ref[:, :, pl.ds(c * CH, CH), :]

        def rs_rdma(t):
            return pltpu.make_async_remote_copy(
                src_ref=comm_ref.at[t],
                dst_ref=comm_ref.at[t + 1],
                send_sem=rs_send.at[t],
                recv_sem=rs_recv.at[t + 1],
                device_id=(right,),
                device_id_type=pl.DeviceIdType.MESH,
            )

        comm_ref[0] = chunk(my % N_DEV)
        rs_rdma(0).start()
        for t in range(1, N_DEV):
            rs_rdma(t - 1).wait_recv()
            comm_ref[t] = comm_ref[t] + chunk((my - t) % N_DEV)
            if t < N_DEV - 1:
                rs_rdma(t).start()

        fin = N_DEV - 1
        for b in range(B):
            for h in range(Hq):
                nck_ref[b * CH:(b + 1) * CH, h * Dh:(h + 1) * Dh] = (
                    comm_ref[fin, b, h] / comm_ref[fin, b, Hq, :, h:h + 1])
        comm2_ref[0] = jnp.dot(nck_ref[...], wo_ref[...],
                               preferred_element_type=jnp.float32)

        def ag_rdma(g):
            return pltpu.make_async_remote_copy(
                src_ref=comm2_ref.at[g],
                dst_ref=comm2_ref.at[g + 1],
                send_sem=ag_send.at[g],
                recv_sem=ag_recv.at[g + 1],
                device_id=(right,),
                device_id_type=pl.DeviceIdType.MESH,
            )

        def store_out(g):
            c_o = (my - g + 1) % N_DEV
            for b in range(B):
                out_ref[b, pl.ds(c_o * CH, CH), :] = comm2_ref[
                    g, b * CH:(b + 1) * CH, :]

        ag_rdma(0).start()
        store_out(0)
        for g in range(1, N_DEV):
            ag_rdma(g - 1).wait_recv()
            if g < N_DEV - 1:
                ag_rdma(g).start()
            store_out(g)

        for t in range(N_DEV - 1):
            rs_rdma(t).wait_send()
            ag_rdma(t).wait_send()

    return pl.pallas_call(
        body,
        out_shape=jax.ShapeDtypeStruct((B, Sq, D), jnp.float32),
        in_specs=[
            pl.BlockSpec(memory_space=pltpu.MemorySpace.VMEM),
            pl.BlockSpec(memory_space=pltpu.MemorySpace.VMEM),
            pl.BlockSpec(memory_space=pltpu.MemorySpace.VMEM),
            pl.BlockSpec(memory_space=pltpu.MemorySpace.HBM),
            pl.BlockSpec(memory_space=pltpu.MemorySpace.HBM),
        ],
        out_specs=pl.BlockSpec(memory_space=pltpu.MemorySpace.VMEM),
        scratch_shapes=[
            pltpu.VMEM((B * Sq, D), jnp.float32),
            pltpu.VMEM((B, HL, Sq, Dh), jnp.float32),
            pltpu.VMEM((B * CH, D), jnp.float32),
            pltpu.VMEM((N_DEV, B, HL, CH, Dh), jnp.float32),
            pltpu.VMEM((N_DEV, B * CH, D), jnp.float32),
            pltpu.VMEM((2, SKV, Hq, Dh), jnp.float32),
            pltpu.VMEM((2, SKV, Hq, Dh), jnp.float32),
            pltpu.SemaphoreType.DMA((N_DEV,)),
            pltpu.SemaphoreType.DMA((N_DEV,)),
            pltpu.SemaphoreType.DMA((N_DEV,)),
            pltpu.SemaphoreType.DMA((N_DEV,)),
            pltpu.SemaphoreType.DMA((2, 2)),
        ],
        compiler_params=pltpu.CompilerParams(
            collective_id=0,
            vmem_limit_bytes=128 * 1024 * 1024,
        ),
    )(x, Wq, Wo, K_ext, V_ext)


# device time: 90227 ns/iter; 9.4514x vs baseline; 2.1075x over previous
import jax
import jax.numpy as jnp
from jax import lax
from jax.experimental import pallas as pl
from jax.experimental.pallas import tpu as pltpu

N_DEV = 16
B, Sq, D, Hq, Dh = 4, 256, 1024, 8, 128
SKV = 1024
SCALE = 0.08838834764831843
HL = Hq + 1
QCH = 64
QPB = 4


def kernel(x, Wq, Wo, K_ext, V_ext):
    def body(x_ref, wq_ref, wo_ref, k_ref, v_ref, out_ref,
             qs_ref, part_ref, nck_ref, comm_ref, comm2_ref, kbuf, vbuf,
             rs_send, rs_recv, ag_send, ag_recv, local_sems):
        my = lax.axis_index("i")
        g0 = my // QPB
        c_my = (my + 1) % N_DEV

        barrier = pltpu.get_barrier_semaphore()
        for o in range(1, N_DEV):
            pl.semaphore_signal(barrier, inc=1,
                                device_id=((my + o) % N_DEV,),
                                device_id_type=pl.DeviceIdType.MESH)
        pl.semaphore_wait(barrier, N_DEV - 1)

        def kv_copies(b, slot):
            cp_k = pltpu.make_async_copy(
                k_ref.at[b], kbuf.at[slot], local_sems.at[slot, 0])
            cp_v = pltpu.make_async_copy(
                v_ref.at[b], vbuf.at[slot], local_sems.at[slot, 1])
            return cp_k, cp_v

        def start_kv(b, slot):
            cp_k, cp_v = kv_copies(b, slot)
            cp_k.start()
            cp_v.start()

        start_kv(g0, 0)

        qs_ref[...] = jnp.dot(
            x_ref[...].reshape(B * Sq, D), wq_ref[...],
            preferred_element_type=jnp.float32) * SCALE

        def rs_send_rdma(c):
            return pltpu.make_async_remote_copy(
                src_ref=part_ref.at[c],
                dst_ref=comm_ref.at[my],
                send_sem=rs_send.at[c],
                recv_sem=rs_recv.at[my],
                device_id=((c - 1) % N_DEV,),
                device_id_type=pl.DeviceIdType.MESH,
            )

        def rs_recv_rdma(origin):
            return pltpu.make_async_remote_copy(
                src_ref=part_ref.at[0],
                dst_ref=comm_ref.at[origin],
                send_sem=rs_send.at[0],
                recv_sem=rs_recv.at[origin],
                device_id=(origin,),
                device_id_type=pl.DeviceIdType.MESH,
            )

        def jbody(j, carry):
            b = (g0 - j) % B
            slot = j % 2

            @pl.when(j + 1 < B)
            def _():
                start_kv((g0 - j - 1) % B, (j + 1) % 2)

            cp_k, cp_v = kv_copies(0, slot)
            cp_k.wait()
            cp_v.wait()
            for h in range(Hq):
                q = qs_ref[pl.ds(b * Sq, Sq), h * Dh:(h + 1) * Dh]
                s = lax.dot_general(q, kbuf[slot, :, h, :],
                                    (((1,), (1,)), ((), ())),
                                    preferred_element_type=jnp.float32)
                p = jnp.exp(s)
                ov = jnp.dot(p, vbuf[slot, :, h, :],
                             preferred_element_type=jnp.float32)
                lv = jnp.sum(p, axis=1, keepdims=True)
                for qq in range(QPB):
                    rows = slice(qq * QCH, (qq + 1) * QCH)
                    part_ref[b * QPB + qq, h] = ov[rows].astype(jnp.bfloat16)
                    part_ref[b * QPB + qq, Hq, :, h:h + 1] = (
                        lv[rows].astype(jnp.bfloat16))
            for qq in range(QPB):
                c = b * QPB + qq

                @pl.when(c != c_my)
                def _():
                    rs_send_rdma(c).start()

            return carry

        lax.fori_loop(0, B, jbody, 0)

        comm_ref[my] = part_ref[c_my]
        for o in range(1, N_DEV):
            rs_recv_rdma((my + o) % N_DEV).wait_recv()
        fin = jnp.sum(comm_ref[...].astype(jnp.float32), axis=0)

        for h in range(Hq):
            nck_ref[:, h * Dh:(h + 1) * Dh] = fin[h] / fin[Hq, :, h:h + 1]
        res = jnp.dot(nck_ref[...], wo_ref[...],
                      preferred_element_type=jnp.float32)
        comm2_ref[my] = res.astype(jnp.bfloat16)
        out_ref[c_my // QPB, pl.ds((c_my % QPB) * QCH, QCH), :] = res

        def ag_rdma(tgt):
            return pltpu.make_async_remote_copy(
                src_ref=comm2_ref.at[my],
                dst_ref=comm2_ref.at[my],
                send_sem=ag_send.at[tgt],
                recv_sem=ag_recv.at[my],
                device_id=(tgt,),
                device_id_type=pl.DeviceIdType.MESH,
            )

        def ag_recv_rdma(origin):
            return pltpu.make_async_remote_copy(
                src_ref=comm2_ref.at[origin],
                dst_ref=comm2_ref.at[origin],
                send_sem=ag_send.at[origin],
                recv_sem=ag_recv.at[origin],
                device_id=(origin,),
                device_id_type=pl.DeviceIdType.MESH,
            )

        for o in range(1, N_DEV):
            ag_rdma((my + o) % N_DEV).start()
        for o in range(1, N_DEV):
            origin = (my + o) % N_DEV
            ag_recv_rdma(origin).wait_recv()
            c_o = (origin + 1) % N_DEV
            out_ref[c_o // QPB, pl.ds((c_o % QPB) * QCH, QCH), :] = (
                comm2_ref[origin].astype(jnp.float32))

        for o in range(1, N_DEV):
            rs_send_rdma((c_my + o) % N_DEV).wait_send()
            ag_rdma((my + o) % N_DEV).wait_send()

    return pl.pallas_call(
        body,
        out_shape=jax.ShapeDtypeStruct((B, Sq, D), jnp.float32),
        in_specs=[
            pl.BlockSpec(memory_space=pltpu.MemorySpace.VMEM),
            pl.BlockSpec(memory_space=pltpu.MemorySpace.VMEM),
            pl.BlockSpec(memory_space=pltpu.MemorySpace.VMEM),
            pl.BlockSpec(memory_space=pltpu.MemorySpace.HBM),
            pl.BlockSpec(memory_space=pltpu.MemorySpace.HBM),
        ],
        out_specs=pl.BlockSpec(memory_space=pltpu.MemorySpace.VMEM),
        scratch_shapes=[
            pltpu.VMEM((B * Sq, D), jnp.float32),
            pltpu.VMEM((N_DEV, HL, QCH, Dh), jnp.bfloat16),
            pltpu.VMEM((QCH, D), jnp.float32),
            pltpu.VMEM((N_DEV, HL, QCH, Dh), jnp.bfloat16),
            pltpu.VMEM((N_DEV, QCH, D), jnp.bfloat16),
            pltpu.VMEM((2, SKV, Hq, Dh), jnp.float32),
            pltpu.VMEM((2, SKV, Hq, Dh), jnp.float32),
            pltpu.SemaphoreType.DMA((N_DEV,)),
            pltpu.SemaphoreType.DMA((N_DEV,)),
            pltpu.SemaphoreType.DMA((N_DEV,)),
            pltpu.SemaphoreType.DMA((N_DEV,)),
            pltpu.SemaphoreType.DMA((2, 2)),
        ],
        compiler_params=pltpu.CompilerParams(
            collective_id=0,
            vmem_limit_bytes=128 * 1024 * 1024,
        ),
    )(x, Wq, Wo, K_ext, V_ext)
